# baseline (speedup 1.0000x reference)
import functools

import jax
import jax.numpy as jnp
from jax.experimental import pallas as pl

N = 4096
D_IN = 256
D_H = 128
HEADS = 8
HEAD_DIM = D_H // HEADS

_BN = 512
_BQ = 256


def _pre_kernel(x_ref, wz_ref, bz_ref, wh_ref, bh_ref, wqkv_ref, bqkv_ref,
                h_ref, q_ref, k_ref, v_ref):
    xb = x_ref[...]
    z = jax.nn.sigmoid(
        jnp.dot(xb, wz_ref[...], preferred_element_type=jnp.float32)
        + bz_ref[...])
    ht = jnp.tanh(
        jnp.dot(xb, wh_ref[...], preferred_element_type=jnp.float32)
        + bh_ref[...])
    hb = (1.0 - z) * ht
    qkv = (jnp.dot(hb, wqkv_ref[...], preferred_element_type=jnp.float32)
           + bqkv_ref[...])
    h_ref[...] = hb
    q_ref[...] = qkv[:, :D_H]
    k_ref[...] = qkv[:, D_H:2 * D_H]
    v_ref[...] = qkv[:, 2 * D_H:]


def _attn_kernel(q_ref, h_ref, k_ref, v_ref, wo_ref, bo_ref, lng_ref, lnb_ref,
                 w1_ref, b1_ref, w2_ref, b2_ref, out_ref):
    qb = q_ref[...]
    kf = k_ref[...].astype(jnp.bfloat16)
    vf = v_ref[...].astype(jnp.bfloat16)
    nkeys = kf.shape[0]
    ones_col = jnp.ones((nkeys, 1), jnp.bfloat16)
    scale = 1.4426950408889634 / (HEAD_DIM ** 0.5)
    ctx_parts = []
    for h in range(HEADS):
        sl = slice(h * HEAD_DIM, (h + 1) * HEAD_DIM)
        qh = (qb[:, sl] * scale).astype(jnp.bfloat16)
        kh = kf[:, sl]
        s = jax.lax.dot_general(qh, kh, (((1,), (1,)), ((), ())),
                                preferred_element_type=jnp.float32)
        e = jnp.exp2(jnp.minimum(s, 86.0)).astype(jnp.bfloat16)
        vaug = jnp.concatenate([vf[:, sl], ones_col], axis=1)
        cs = jnp.dot(e, vaug, preferred_element_type=jnp.float32)
        ctx_parts.append(cs[:, :HEAD_DIM] / cs[:, HEAD_DIM:])
    ctx = jnp.concatenate(ctx_parts, axis=1)
    att = (jnp.dot(ctx, wo_ref[...], preferred_element_type=jnp.float32)
           + bo_ref[...])
    hb = h_ref[...]
    y = hb + att
    mu = jnp.mean(y, axis=-1, keepdims=True)
    yc = y - mu
    var = jnp.mean(yc * yc, axis=-1, keepdims=True)
    ln = yc * jax.lax.rsqrt(var + 1e-5) * lng_ref[...] + lnb_ref[...]
    h2 = hb + ln
    t = jnp.maximum(
        jnp.dot(h2, w1_ref[...], preferred_element_type=jnp.float32)
        + b1_ref[...], 0.0)
    out_ref[...] = (jnp.dot(t, w2_ref[...], preferred_element_type=jnp.float32)
                    + b2_ref[...])


@functools.partial(jax.jit, static_argnames=())
def kernel(x, edge_index, W_z, b_z, W_r, b_r, W_h, b_h, in_proj_w, in_proj_b,
           out_proj_w, out_proj_b, ln_g, ln_b, fc1_w, fc1_b, fc2_w, fc2_b):
    del edge_index, W_r, b_r
    n = x.shape[0]

    wz = (W_z[0, 0] + W_z[1, 0])[:D_IN]
    wh = (W_h[0, 0] + W_h[1, 0])[:D_IN]
    wqkv = in_proj_w.T
    row = lambda v: v.reshape(1, -1)

    grid_pre = n // _BN
    h_, q_, k_, v_ = pl.pallas_call(
        _pre_kernel,
        grid=(grid_pre,),
        in_specs=[
            pl.BlockSpec((_BN, D_IN), lambda i: (i, 0)),
            pl.BlockSpec((D_IN, D_H), lambda i: (0, 0)),
            pl.BlockSpec((1, D_H), lambda i: (0, 0)),
            pl.BlockSpec((D_IN, D_H), lambda i: (0, 0)),
            pl.BlockSpec((1, D_H), lambda i: (0, 0)),
            pl.BlockSpec((D_H, 3 * D_H), lambda i: (0, 0)),
            pl.BlockSpec((1, 3 * D_H), lambda i: (0, 0)),
        ],
        out_specs=[
            pl.BlockSpec((_BN, D_H), lambda i: (i, 0)),
            pl.BlockSpec((_BN, D_H), lambda i: (i, 0)),
            pl.BlockSpec((_BN, D_H), lambda i: (i, 0)),
            pl.BlockSpec((_BN, D_H), lambda i: (i, 0)),
        ],
        out_shape=[jax.ShapeDtypeStruct((n, D_H), jnp.float32)] * 4,
    )(x, wz, row(b_z), wh, row(b_h), wqkv, row(in_proj_b))

    grid_attn = n // _BQ
    out = pl.pallas_call(
        _attn_kernel,
        grid=(grid_attn,),
        in_specs=[
            pl.BlockSpec((_BQ, D_H), lambda i: (i, 0)),
            pl.BlockSpec((_BQ, D_H), lambda i: (i, 0)),
            pl.BlockSpec((n, D_H), lambda i: (0, 0)),
            pl.BlockSpec((n, D_H), lambda i: (0, 0)),
            pl.BlockSpec((D_H, D_H), lambda i: (0, 0)),
            pl.BlockSpec((1, D_H), lambda i: (0, 0)),
            pl.BlockSpec((1, D_H), lambda i: (0, 0)),
            pl.BlockSpec((1, D_H), lambda i: (0, 0)),
            pl.BlockSpec((D_H, D_H), lambda i: (0, 0)),
            pl.BlockSpec((1, D_H), lambda i: (0, 0)),
            pl.BlockSpec((D_H, 1), lambda i: (0, 0)),
            pl.BlockSpec((1, 1), lambda i: (0, 0)),
        ],
        out_specs=pl.BlockSpec((_BQ, 1), lambda i: (i, 0)),
        out_shape=jax.ShapeDtypeStruct((n, 1), jnp.float32),
    )(q_, h_, k_, v_, out_proj_w.T, row(out_proj_b), row(ln_g), row(ln_b),
      fc1_w.T, row(fc1_b), fc2_w.T, fc2_b.reshape(1, 1))
    return out


# speedup vs baseline: 1.1512x; 1.1512x over previous
import functools

import jax
import jax.numpy as jnp
from jax.experimental import pallas as pl
from jax.experimental.pallas import tpu as pltpu

N = 4096
D_IN = 256
D_H = 128
HEADS = 8
HEAD_DIM = D_H // HEADS

_BQ = 512


def _fused_kernel(x_ref, wz_ref, bz_ref, wh_ref, bh_ref, wqkv_ref, bqkv_ref,
                  wo_ref, bo_ref, lng_ref, lnb_ref, w1_ref, b1_ref, w2_ref,
                  b2_ref, out_ref, h_s, q_s, k_s, v_s):
    i = pl.program_id(0)

    @pl.when(i == 0)
    def _prologue():
        xb = x_ref[...]
        z = jax.nn.sigmoid(
            jnp.dot(xb, wz_ref[...], preferred_element_type=jnp.float32)
            + bz_ref[...])
        ht = jnp.tanh(
            jnp.dot(xb, wh_ref[...], preferred_element_type=jnp.float32)
            + bh_ref[...])
        hb = (1.0 - z) * ht
        qkv = (jnp.dot(hb, wqkv_ref[...], preferred_element_type=jnp.float32)
               + bqkv_ref[...])
        h_s[...] = hb
        q_s[...] = qkv[:, :D_H]
        k_s[...] = qkv[:, D_H:2 * D_H].astype(jnp.bfloat16)
        v_s[...] = qkv[:, 2 * D_H:].astype(jnp.bfloat16)

    rows = pl.ds(i * _BQ, _BQ)
    qb = q_s[rows, :]
    kf = k_s[...]
    vf = v_s[...]
    ones_col = jnp.ones((N, 1), jnp.bfloat16)
    scale = 1.4426950408889634 / (HEAD_DIM ** 0.5)
    ctx_parts = []
    for h in range(HEADS):
        sl = slice(h * HEAD_DIM, (h + 1) * HEAD_DIM)
        qh = (qb[:, sl] * scale).astype(jnp.bfloat16)
        s = jax.lax.dot_general(qh, kf[:, sl], (((1,), (1,)), ((), ())),
                                preferred_element_type=jnp.float32)
        e = jnp.exp2(jnp.minimum(s, 86.0)).astype(jnp.bfloat16)
        vaug = jnp.concatenate([vf[:, sl], ones_col], axis=1)
        cs = jnp.dot(e, vaug, preferred_element_type=jnp.float32)
        ctx_parts.append(cs[:, :HEAD_DIM] / cs[:, HEAD_DIM:])
    ctx = jnp.concatenate(ctx_parts, axis=1)
    att = (jnp.dot(ctx, wo_ref[...], preferred_element_type=jnp.float32)
           + bo_ref[...])
    hb = h_s[rows, :]
    y = hb + att
    mu = jnp.mean(y, axis=-1, keepdims=True)
    yc = y - mu
    var = jnp.mean(yc * yc, axis=-1, keepdims=True)
    ln = yc * jax.lax.rsqrt(var + 1e-5) * lng_ref[...] + lnb_ref[...]
    h2 = hb + ln
    t = jnp.maximum(
        jnp.dot(h2, w1_ref[...], preferred_element_type=jnp.float32)
        + b1_ref[...], 0.0)
    out_ref[...] = (jnp.dot(t, w2_ref[...], preferred_element_type=jnp.float32)
                    + b2_ref[...])


@functools.partial(jax.jit, static_argnames=())
def kernel(x, edge_index, W_z, b_z, W_r, b_r, W_h, b_h, in_proj_w, in_proj_b,
           out_proj_w, out_proj_b, ln_g, ln_b, fc1_w, fc1_b, fc2_w, fc2_b):
    del edge_index, W_r, b_r
    n = x.shape[0]

    wz = (W_z[0, 0] + W_z[1, 0])[:D_IN]
    wh = (W_h[0, 0] + W_h[1, 0])[:D_IN]
    wqkv = in_proj_w.T
    row = lambda v: v.reshape(1, -1)

    full = lambda shape: pl.BlockSpec(shape, lambda i: (0,) * len(shape))
    grid = n // _BQ
    out = pl.pallas_call(
        _fused_kernel,
        grid=(grid,),
        in_specs=[
            full((n, D_IN)),
            full((D_IN, D_H)),
            full((1, D_H)),
            full((D_IN, D_H)),
            full((1, D_H)),
            full((D_H, 3 * D_H)),
            full((1, 3 * D_H)),
            full((D_H, D_H)),
            full((1, D_H)),
            full((1, D_H)),
            full((1, D_H)),
            full((D_H, D_H)),
            full((1, D_H)),
            full((D_H, 1)),
            full((1, 1)),
        ],
        out_specs=pl.BlockSpec((_BQ, 1), lambda i: (i, 0)),
        out_shape=jax.ShapeDtypeStruct((n, 1), jnp.float32),
        scratch_shapes=[
            pltpu.VMEM((n, D_H), jnp.float32),
            pltpu.VMEM((n, D_H), jnp.float32),
            pltpu.VMEM((n, D_H), jnp.bfloat16),
            pltpu.VMEM((n, D_H), jnp.bfloat16),
        ],
    )(x, wz, row(b_z), wh, row(b_h), wqkv, row(in_proj_b),
      out_proj_w.T, row(out_proj_b), row(ln_g), row(ln_b),
      fc1_w.T, row(fc1_b), fc2_w.T, fc2_b.reshape(1, 1))
    return out
